# baseline (speedup 1.0000x reference)
import functools

import numpy as np
import jax
import jax.numpy as jnp
from jax import lax
from jax.experimental import pallas as pl
from jax.experimental.pallas import tpu as pltpu
from jax.experimental.pallas import tpu_sc as plsc

N = 10000
D = 128
E = 320000

NC = 2
NS = 16
NW = NC * NS

K = 128
NCH = 80
IB = 40
IB_DEG = 40
EPT = K * NCH
E_PAD = NW * EPT


ACC_ROWS = 10240
ZCH = ACC_ROWS // NS // K
RPT = ACC_ROWS // NS

M_BLK = 1000
GRID = N // M_BLK

DEG_W = 16

_PAD_N = E_PAD - E
_PAD_SRC = (np.arange(_PAD_N, dtype=np.int32) * 37) % N
_PAD_DST = N + (np.arange(_PAD_N, dtype=np.int32) % (ACC_ROWS - N))

_MESH = plsc.VectorSubcoreMesh(
    core_axis_name="c", subcore_axis_name="s", num_cores=NC, num_subcores=NS
)


def _fill2d(buf, rows, cols, val):
  vec = jnp.full((16,), val, jnp.float32)

  def body(i, carry):
    for j in range(cols // 16):
      buf[i, pl.ds(j * 16, 16)] = vec
    return carry

  lax.fori_loop(0, rows, body, 0)


@functools.partial(
    pl.kernel,
    out_type=jax.ShapeDtypeStruct((NC, ACC_ROWS, D), jnp.float32),
    mesh=_MESH,
    scratch_types=[
        pltpu.VMEM((IB_DEG, K), jnp.int32),
        pltpu.VMEM((K, D), jnp.float32),
        pltpu.VMEM_SHARED((ACC_ROWS, D), jnp.float32),
    ],
)
def _sc_degree(dst_hbm, out_hbm, dst_v, ones_v, acc_sh):
  c = lax.axis_index("c")
  s = lax.axis_index("s")
  wid = c * NS + s

  _fill2d(ones_v, K, D, 0.0)
  for t in range(ZCH):
    pltpu.sync_copy(ones_v, acc_sh.at[pl.ds(s * (ZCH * K) + t * K, K)])
  plsc.subcore_barrier()

  _fill2d(ones_v, K, D, 1.0)

  def blk(b, carry):
    pltpu.sync_copy(dst_hbm.at[wid, pl.ds(b * IB_DEG, IB_DEG)], dst_v)

    def body(j, carry2):
      pltpu.sync_copy(ones_v, acc_sh.at[dst_v.at[j]], add=True)
      return carry2

    lax.fori_loop(0, IB_DEG, body, 0)
    return carry

  lax.fori_loop(0, NCH // IB_DEG, blk, 0)
  plsc.subcore_barrier()

  pltpu.sync_copy(acc_sh.at[pl.ds(s * RPT, RPT)], out_hbm.at[c, pl.ds(s * RPT, RPT)])


@functools.partial(
    pl.kernel,
    out_type=jax.ShapeDtypeStruct((NC, ACC_ROWS, D), jnp.float32),
    mesh=_MESH,
    scratch_types=[
        pltpu.VMEM((IB, K), jnp.int32),
        pltpu.VMEM((IB, K), jnp.int32),
        pltpu.VMEM((K, D), jnp.float32),
        pltpu.VMEM((K, D), jnp.float32),
        pltpu.SemaphoreType.DMA,
        pltpu.SemaphoreType.DMA,
        pltpu.VMEM_SHARED((ACC_ROWS, D), jnp.float32),
    ],
)
def _sc_prop(y_hbm, src_hbm, dst_hbm, out_hbm,
             src_v, dst_v, rows0, rows1, sem0, sem1, acc_sh):
  c = lax.axis_index("c")
  s = lax.axis_index("s")
  wid = c * NS + s

  _fill2d(rows0, K, D, 0.0)
  for t in range(ZCH):
    pltpu.sync_copy(rows0, acc_sh.at[pl.ds(s * (ZCH * K) + t * K, K)])
  plsc.subcore_barrier()

  def run_edges(src_hbm, dst_hbm, nch):
    def blk(b, carry):
      pltpu.sync_copy(src_hbm.at[wid, pl.ds(b * IB, IB)], src_v)
      pltpu.sync_copy(dst_hbm.at[wid, pl.ds(b * IB, IB)], dst_v)
      pltpu.async_copy(y_hbm.at[src_v.at[0]], rows0, sem0)

      def body(i, carry2):
        j0 = 2 * i
        pltpu.async_copy(y_hbm.at[src_v.at[j0 + 1]], rows1, sem1)
        pltpu.make_async_copy(y_hbm.at[src_v.at[j0]], rows0, sem0).wait()
        pltpu.sync_copy(rows0, acc_sh.at[dst_v.at[j0]], add=True)

        @pl.when(j0 + 2 < IB)
        def _():
          pltpu.async_copy(y_hbm.at[src_v.at[j0 + 2]], rows0, sem0)

        pltpu.make_async_copy(y_hbm.at[src_v.at[j0 + 1]], rows1, sem1).wait()
        pltpu.sync_copy(rows1, acc_sh.at[dst_v.at[j0 + 1]], add=True)
        return carry2

      lax.fori_loop(0, IB // 2, body, 0)
      return carry

    lax.fori_loop(0, nch // IB, blk, 0)

  run_edges(src_hbm, dst_hbm, NCH)

  plsc.subcore_barrier()

  pltpu.sync_copy(acc_sh.at[pl.ds(s * RPT, RPT)], out_hbm.at[c, pl.ds(s * RPT, RPT)])


def _dinv_block(deg_ref):
  d = deg_ref[0, :, 0:1] + deg_ref[1, :, 0:1] + 1.0
  return lax.rsqrt(jnp.maximum(d, 1.0))


def _tc_mm_body(x_ref, wp_ref, bp_ref, w1_ref, xp_ref, xw1_ref):
  x = x_ref[...]
  xp_ref[...] = jnp.maximum(
      jnp.dot(x, wp_ref[...], preferred_element_type=jnp.float32) + bp_ref[...],
      0.0)
  xw1_ref[...] = jnp.dot(x, w1_ref[...], preferred_element_type=jnp.float32)


def _tc_scale_body(xw1_ref, deg_ref, y1_ref):
  y1_ref[...] = _dinv_block(deg_ref) * xw1_ref[...]


def _tc_parts_body(xp_ref, w2a_ref, w3a_ref, p2_ref, p3a_ref):
  xp = xp_ref[...]
  p2_ref[...] = jnp.dot(xp, w2a_ref[...], preferred_element_type=jnp.float32)
  p3a_ref[...] = jnp.dot(xp, w3a_ref[...], preferred_element_type=jnp.float32)


def _tc_part3b_body(p3a_ref, h1_ref, w3b_ref, p3_ref):
  p3_ref[...] = p3a_ref[...] + jnp.dot(
      h1_ref[...], w3b_ref[...], preferred_element_type=jnp.float32)


def _tc_fuse1_body(acc_ref, y1_ref, deg_ref, b1_ref, p2_ref, w2b_ref,
                   h1_ref, y2_ref):
  dinv = _dinv_block(deg_ref)
  h1 = jnp.maximum(
      dinv * (acc_ref[0] + acc_ref[1] + y1_ref[...]) + b1_ref[...], 0.0)
  h1_ref[...] = h1
  y2_ref[...] = dinv * (p2_ref[...] + jnp.dot(
      h1, w2b_ref[...], preferred_element_type=jnp.float32))


def _tc_fuse2_body(acc_ref, y2_ref, deg_ref, b2_ref, p3_ref, w3c_ref,
                   h2_ref, y3_ref):
  dinv = _dinv_block(deg_ref)
  h2 = jnp.maximum(
      dinv * (acc_ref[0] + acc_ref[1] + y2_ref[...]) + b2_ref[...], 0.0)
  h2_ref[...] = h2
  y3_ref[...] = dinv * (p3_ref[...] + jnp.dot(
      h2, w3c_ref[...], preferred_element_type=jnp.float32))


def _tc_fuse3_body(acc_ref, y3_ref, deg_ref, b3_ref, h3_ref):
  dinv = _dinv_block(deg_ref)
  h3_ref[...] = jnp.maximum(
      dinv * (acc_ref[0] + acc_ref[1] + y3_ref[...]) + b3_ref[...], 0.0)


def _row_spec(shape_tail):
  return pl.BlockSpec((M_BLK,) + shape_tail, lambda i: (i,) + (0,) * len(shape_tail))


def _acc_spec(tail):
  return pl.BlockSpec((NC, M_BLK, tail), lambda i: (0, i, 0))


_DEG_SPEC = pl.BlockSpec((NC, M_BLK, D), lambda i: (0, i, 0))


def _full_spec(shape):
  return pl.BlockSpec(shape, lambda i: (0,) * len(shape))


_F32 = jnp.float32


def _mm2(body, ins, specs, n_out):
  return pl.pallas_call(
      body,
      grid=(GRID,),
      in_specs=specs,
      out_specs=[_row_spec((D,))] * n_out if n_out > 1 else _row_spec((D,)),
      out_shape=[jax.ShapeDtypeStruct((N, D), _F32)] * n_out if n_out > 1
      else jax.ShapeDtypeStruct((N, D), _F32),
  )(*ins)


def _tc_mm(x, W_proj, b_proj, W1):
  return _mm2(_tc_mm_body, (x, W_proj, b_proj, W1),
              [_row_spec((D,)), _full_spec((D, D)), _full_spec((1, D)),
               _full_spec((D, D))], 2)


def _tc_scale(xw1, deg2):
  return _mm2(_tc_scale_body, (xw1, deg2),
              [_row_spec((D,)), _DEG_SPEC], 1)


def _tc_parts(xp, W2a, W3a):
  return _mm2(_tc_parts_body, (xp, W2a, W3a),
              [_row_spec((D,)), _full_spec((D, D)), _full_spec((D, D))], 2)


def _tc_part3b(p3a, h1, W3b):
  return _mm2(_tc_part3b_body, (p3a, h1, W3b),
              [_row_spec((D,)), _row_spec((D,)), _full_spec((D, D))], 1)


def _tc_fuse1(acc, y1, deg2, b1, p2, W2b):
  return _mm2(_tc_fuse1_body, (acc, y1, deg2, b1, p2, W2b),
              [_acc_spec(D), _row_spec((D,)), _DEG_SPEC,
               _full_spec((1, D)), _row_spec((D,)), _full_spec((D, D))], 2)


def _tc_fuse2(acc, y2, deg2, b2, p3, W3c):
  return _mm2(_tc_fuse2_body, (acc, y2, deg2, b2, p3, W3c),
              [_acc_spec(D), _row_spec((D,)), _DEG_SPEC,
               _full_spec((1, D)), _row_spec((D,)), _full_spec((D, D))], 2)


def _tc_fuse3(acc, y3, deg2, b3):
  return _mm2(_tc_fuse3_body, (acc, y3, deg2, b3),
              [_acc_spec(D), _row_spec((D,)), _DEG_SPEC,
               _full_spec((1, D))], 1)


def kernel(x, edge_index, W_proj, b_proj, W1, b1, W2, b2, W3, b3):
  src_p = jnp.concatenate([edge_index[0], _PAD_SRC])
  dst_p = jnp.concatenate([edge_index[1], _PAD_DST])
  src3 = src_p.reshape(NW, NCH, K)
  dst3 = dst_p.reshape(NW, NCH, K)

  b_proj2 = b_proj.reshape(1, D)
  b1_2 = b1.reshape(1, D)
  b2_2 = b2.reshape(1, D)
  b3_2 = b3.reshape(1, D)

  deg2 = _sc_degree(dst3)
  xp, xw1 = _tc_mm(x, W_proj, b_proj2, W1)
  y1 = _tc_scale(xw1, deg2)
  acc1 = _sc_prop(y1, src3, dst3)
  p2, p3a = _tc_parts(xp, W2[:D], W3[:D])
  h1, y2 = _tc_fuse1(acc1, y1, deg2, b1_2, p2, W2[D:])
  acc2 = _sc_prop(y2, src3, dst3)
  p3 = _tc_part3b(p3a, h1, W3[D:2 * D])
  h2, y3 = _tc_fuse2(acc2, y2, deg2, b2_2, p3, W3[2 * D:])
  acc3 = _sc_prop(y3, src3, dst3)
  return _tc_fuse3(acc3, y3, deg2, b3_2)


# speedup vs baseline: 1.0454x; 1.0454x over previous
import functools

import numpy as np
import jax
import jax.numpy as jnp
from jax import lax
from jax.experimental import pallas as pl
from jax.experimental.pallas import tpu as pltpu
from jax.experimental.pallas import tpu_sc as plsc

N = 10000
D = 128
E = 320000

NC = 2
NS = 16
NW = NC * NS

K = 128
NCH = 80
IB = 40
IB_DEG = 40
EPT = K * NCH
E_PAD = NW * EPT


ACC_ROWS = 10240
ZCH = ACC_ROWS // NS // K
RPT = ACC_ROWS // NS

M_BLK = 1000
GRID = N // M_BLK

DEG_W = 16

K2 = 64
NCH2 = E_PAD // (NW * K2)
IB2 = 40
NBUF = 4

_PAD_N = E_PAD - E
_PAD_SRC = (np.arange(_PAD_N, dtype=np.int32) * 37) % N
_PAD_DST = N + (np.arange(_PAD_N, dtype=np.int32) % (ACC_ROWS - N))

_MESH = plsc.VectorSubcoreMesh(
    core_axis_name="c", subcore_axis_name="s", num_cores=NC, num_subcores=NS
)


def _fill2d(buf, rows, cols, val):
  vec = jnp.full((16,), val, jnp.float32)

  def body(i, carry):
    for j in range(cols // 16):
      buf[i, pl.ds(j * 16, 16)] = vec
    return carry

  lax.fori_loop(0, rows, body, 0)


@functools.partial(
    pl.kernel,
    out_type=jax.ShapeDtypeStruct((NC, ACC_ROWS, D), jnp.float32),
    mesh=_MESH,
    scratch_types=[
        pltpu.VMEM((IB_DEG, K), jnp.int32),
        pltpu.VMEM((K, D), jnp.float32),
        pltpu.VMEM_SHARED((ACC_ROWS, D), jnp.float32),
    ],
)
def _sc_degree(dst_hbm, out_hbm, dst_v, ones_v, acc_sh):
  c = lax.axis_index("c")
  s = lax.axis_index("s")
  wid = c * NS + s

  _fill2d(ones_v, K, D, 0.0)
  for t in range(ZCH):
    pltpu.sync_copy(ones_v, acc_sh.at[pl.ds(s * (ZCH * K) + t * K, K)])
  plsc.subcore_barrier()

  _fill2d(ones_v, K, D, 1.0)

  def blk(b, carry):
    pltpu.sync_copy(dst_hbm.at[wid, pl.ds(b * IB_DEG, IB_DEG)], dst_v)

    def body(j, carry2):
      pltpu.sync_copy(ones_v, acc_sh.at[dst_v.at[j]], add=True)
      return carry2

    lax.fori_loop(0, IB_DEG, body, 0)
    return carry

  lax.fori_loop(0, NCH // IB_DEG, blk, 0)
  plsc.subcore_barrier()

  pltpu.sync_copy(acc_sh.at[pl.ds(s * RPT, RPT)], out_hbm.at[c, pl.ds(s * RPT, RPT)])


@functools.partial(
    pl.kernel,
    out_type=jax.ShapeDtypeStruct((NC, ACC_ROWS, D), jnp.float32),
    mesh=_MESH,
    scratch_types=[
        pltpu.VMEM((IB2, K2), jnp.int32),
        pltpu.VMEM((IB2, K2), jnp.int32),
        [pltpu.VMEM((K2, D), jnp.float32)] * NBUF,
        [pltpu.SemaphoreType.DMA] * NBUF,
        pltpu.VMEM_SHARED((ACC_ROWS, D), jnp.float32),
    ],
)
def _sc_prop(y_hbm, src_hbm, dst_hbm, out_hbm,
             src_v, dst_v, rows, sems, acc_sh):
  c = lax.axis_index("c")
  s = lax.axis_index("s")
  wid = c * NS + s

  _fill2d(rows[0], K2, D, 0.0)
  for t in range(ACC_ROWS // NS // K2):
    pltpu.sync_copy(rows[0], acc_sh.at[pl.ds(s * RPT + t * K2, K2)])
  plsc.subcore_barrier()

  def blk(b, carry):
    pltpu.sync_copy(src_hbm.at[wid, pl.ds(b * IB2, IB2)], src_v)
    pltpu.sync_copy(dst_hbm.at[wid, pl.ds(b * IB2, IB2)], dst_v)
    for t in range(NBUF):
      pltpu.async_copy(y_hbm.at[src_v.at[t]], rows[t], sems[t])

    def body(i, carry2):
      j0 = i * NBUF
      for t in range(NBUF):
        j = j0 + t
        pltpu.make_async_copy(y_hbm.at[src_v.at[j]], rows[t], sems[t]).wait()
        pltpu.sync_copy(rows[t], acc_sh.at[dst_v.at[j]], add=True)

        @pl.when(j + NBUF < IB2)
        def _():
          pltpu.async_copy(y_hbm.at[src_v.at[j + NBUF]], rows[t], sems[t])
      return carry2

    lax.fori_loop(0, IB2 // NBUF, body, 0)
    return carry

  lax.fori_loop(0, NCH2 // IB2, blk, 0)
  plsc.subcore_barrier()

  pltpu.sync_copy(acc_sh.at[pl.ds(s * RPT, RPT)], out_hbm.at[c, pl.ds(s * RPT, RPT)])


def _dinv_block(deg_ref):
  d = deg_ref[0, :, 0:1] + deg_ref[1, :, 0:1] + 1.0
  return lax.rsqrt(jnp.maximum(d, 1.0))


def _tc_mm_body(x_ref, wp_ref, bp_ref, w1_ref, xp_ref, xw1_ref):
  x = x_ref[...]
  xp_ref[...] = jnp.maximum(
      jnp.dot(x, wp_ref[...], preferred_element_type=jnp.float32) + bp_ref[...],
      0.0)
  xw1_ref[...] = jnp.dot(x, w1_ref[...], preferred_element_type=jnp.float32)


def _tc_scale_body(xw1_ref, deg_ref, y1_ref):
  y1_ref[...] = _dinv_block(deg_ref) * xw1_ref[...]


def _tc_parts_body(xp_ref, w2a_ref, w3a_ref, p2_ref, p3a_ref):
  xp = xp_ref[...]
  p2_ref[...] = jnp.dot(xp, w2a_ref[...], preferred_element_type=jnp.float32)
  p3a_ref[...] = jnp.dot(xp, w3a_ref[...], preferred_element_type=jnp.float32)


def _tc_part3b_body(p3a_ref, h1_ref, w3b_ref, p3_ref):
  p3_ref[...] = p3a_ref[...] + jnp.dot(
      h1_ref[...], w3b_ref[...], preferred_element_type=jnp.float32)


def _tc_fuse1_body(acc_ref, y1_ref, deg_ref, b1_ref, p2_ref, w2b_ref,
                   h1_ref, y2_ref):
  dinv = _dinv_block(deg_ref)
  h1 = jnp.maximum(
      dinv * (acc_ref[0] + acc_ref[1] + y1_ref[...]) + b1_ref[...], 0.0)
  h1_ref[...] = h1
  y2_ref[...] = dinv * (p2_ref[...] + jnp.dot(
      h1, w2b_ref[...], preferred_element_type=jnp.float32))


def _tc_fuse2_body(acc_ref, y2_ref, deg_ref, b2_ref, p3_ref, w3c_ref,
                   h2_ref, y3_ref):
  dinv = _dinv_block(deg_ref)
  h2 = jnp.maximum(
      dinv * (acc_ref[0] + acc_ref[1] + y2_ref[...]) + b2_ref[...], 0.0)
  h2_ref[...] = h2
  y3_ref[...] = dinv * (p3_ref[...] + jnp.dot(
      h2, w3c_ref[...], preferred_element_type=jnp.float32))


def _tc_fuse3_body(acc_ref, y3_ref, deg_ref, b3_ref, h3_ref):
  dinv = _dinv_block(deg_ref)
  h3_ref[...] = jnp.maximum(
      dinv * (acc_ref[0] + acc_ref[1] + y3_ref[...]) + b3_ref[...], 0.0)


def _row_spec(shape_tail):
  return pl.BlockSpec((M_BLK,) + shape_tail, lambda i: (i,) + (0,) * len(shape_tail))


def _acc_spec(tail):
  return pl.BlockSpec((NC, M_BLK, tail), lambda i: (0, i, 0))


_DEG_SPEC = pl.BlockSpec((NC, M_BLK, D), lambda i: (0, i, 0))


def _full_spec(shape):
  return pl.BlockSpec(shape, lambda i: (0,) * len(shape))


_F32 = jnp.float32


def _mm2(body, ins, specs, n_out):
  return pl.pallas_call(
      body,
      grid=(GRID,),
      in_specs=specs,
      out_specs=[_row_spec((D,))] * n_out if n_out > 1 else _row_spec((D,)),
      out_shape=[jax.ShapeDtypeStruct((N, D), _F32)] * n_out if n_out > 1
      else jax.ShapeDtypeStruct((N, D), _F32),
  )(*ins)


def _tc_mm(x, W_proj, b_proj, W1):
  return _mm2(_tc_mm_body, (x, W_proj, b_proj, W1),
              [_row_spec((D,)), _full_spec((D, D)), _full_spec((1, D)),
               _full_spec((D, D))], 2)


def _tc_scale(xw1, deg2):
  return _mm2(_tc_scale_body, (xw1, deg2),
              [_row_spec((D,)), _DEG_SPEC], 1)


def _tc_parts(xp, W2a, W3a):
  return _mm2(_tc_parts_body, (xp, W2a, W3a),
              [_row_spec((D,)), _full_spec((D, D)), _full_spec((D, D))], 2)


def _tc_part3b(p3a, h1, W3b):
  return _mm2(_tc_part3b_body, (p3a, h1, W3b),
              [_row_spec((D,)), _row_spec((D,)), _full_spec((D, D))], 1)


def _tc_fuse1(acc, y1, deg2, b1, p2, W2b):
  return _mm2(_tc_fuse1_body, (acc, y1, deg2, b1, p2, W2b),
              [_acc_spec(D), _row_spec((D,)), _DEG_SPEC,
               _full_spec((1, D)), _row_spec((D,)), _full_spec((D, D))], 2)


def _tc_fuse2(acc, y2, deg2, b2, p3, W3c):
  return _mm2(_tc_fuse2_body, (acc, y2, deg2, b2, p3, W3c),
              [_acc_spec(D), _row_spec((D,)), _DEG_SPEC,
               _full_spec((1, D)), _row_spec((D,)), _full_spec((D, D))], 2)


def _tc_fuse3(acc, y3, deg2, b3):
  return _mm2(_tc_fuse3_body, (acc, y3, deg2, b3),
              [_acc_spec(D), _row_spec((D,)), _DEG_SPEC,
               _full_spec((1, D))], 1)


def kernel(x, edge_index, W_proj, b_proj, W1, b1, W2, b2, W3, b3):
  src_p = jnp.concatenate([edge_index[0], _PAD_SRC])
  dst_p = jnp.concatenate([edge_index[1], _PAD_DST])
  src3 = src_p.reshape(NW, NCH2, K2)
  dst3 = dst_p.reshape(NW, NCH2, K2)
  dst3deg = dst_p.reshape(NW, NCH, K)

  b_proj2 = b_proj.reshape(1, D)
  b1_2 = b1.reshape(1, D)
  b2_2 = b2.reshape(1, D)
  b3_2 = b3.reshape(1, D)

  deg2 = _sc_degree(dst3deg)
  xp, xw1 = _tc_mm(x, W_proj, b_proj2, W1)
  y1 = _tc_scale(xw1, deg2)
  acc1 = _sc_prop(y1, src3, dst3)
  p2, p3a = _tc_parts(xp, W2[:D], W3[:D])
  h1, y2 = _tc_fuse1(acc1, y1, deg2, b1_2, p2, W2[D:])
  acc2 = _sc_prop(y2, src3, dst3)
  p3 = _tc_part3b(p3a, h1, W3[D:2 * D])
  h2, y3 = _tc_fuse2(acc2, y2, deg2, b2_2, p3, W3[2 * D:])
  acc3 = _sc_prop(y3, src3, dst3)
  return _tc_fuse3(acc3, y3, deg2, b3_2)
